# Initial kernel scaffold; baseline (speedup 1.0000x reference)
import jax
import jax.numpy as jnp
from jax.experimental import pallas as pl


def kernel(x, edge_index, batch, W1, b1, W2, b2, W3, b3, Wc1, bc1, Wc2, bc2):
    raise NotImplementedError("write your pallas kernel here")


# speedup vs baseline: 18.5968x; 18.5968x over previous
import jax
import jax.numpy as jnp
from jax import lax
from jax.experimental import pallas as pl
from jax.experimental.pallas import tpu as pltpu
from jax.experimental.pallas import tpu_sc as plsc

N = 10000
NPAD = 10112
E = 320000
NC, NS = 2, 16
NW = NC * NS
EPT = E // NW
CH = 128
K = -(-EPT // CH)
EPT_PAD = K * CH
RPS = NPAD // NS

f32 = jnp.float32
i32 = jnp.int32
_HIGH = lax.Precision.HIGHEST


def _mesh():
    return plsc.VectorSubcoreMesh(
        core_axis_name="c", subcore_axis_name="s",
        num_cores=NC, num_subcores=NS)


def _sc_scatter(F):
    def body(table, srcp, dstp, zin, out, idx_s, idx_d, rows, acc, gsem):
        cid = lax.axis_index("c")
        sid = lax.axis_index("s")
        wid = sid * NC + cid
        pltpu.sync_copy(zin.at[pl.ds(sid * RPS, RPS)],
                        acc.at[pl.ds(sid * RPS, RPS)])
        pltpu.sync_copy(srcp.at[wid], idx_s)
        pltpu.sync_copy(dstp.at[wid], idx_d)
        plsc.subcore_barrier()

        def step(j, carry):
            pltpu.async_copy(table.at[idx_s.at[j]], rows, gsem).wait()
            pltpu.sync_copy(rows, acc.at[idx_d.at[j]], add=True)
            return carry

        lax.fori_loop(0, K, step, 0)
        plsc.subcore_barrier()
        pltpu.sync_copy(acc.at[pl.ds(sid * RPS, RPS)],
                        out.at[cid, pl.ds(sid * RPS, RPS)])

    return pl.kernel(
        body,
        out_type=jax.ShapeDtypeStruct((NC, NPAD, F), f32),
        mesh=_mesh(),
        scratch_types=[
            pltpu.VMEM((K, CH), i32),
            pltpu.VMEM((K, CH), i32),
            pltpu.VMEM((CH, F), f32),
            pltpu.VMEM_SHARED((NPAD, F), f32),
            pltpu.SemaphoreType.DMA,
        ],
    )


def _sc_deg():
    def body(dstp, zin, ones_in, out, idx_d, rows, acc):
        cid = lax.axis_index("c")
        sid = lax.axis_index("s")
        wid = sid * NC + cid
        pltpu.sync_copy(zin.at[pl.ds(sid * RPS, RPS)],
                        acc.at[pl.ds(sid * RPS, RPS)])
        pltpu.sync_copy(dstp.at[wid], idx_d)
        pltpu.sync_copy(ones_in, rows)
        plsc.subcore_barrier()

        def step(j, carry):
            pltpu.sync_copy(rows, acc.at[idx_d.at[j]], add=True)
            return carry

        lax.fori_loop(0, K, step, 0)
        plsc.subcore_barrier()
        pltpu.sync_copy(acc.at[pl.ds(sid * RPS, RPS)],
                        out.at[cid, pl.ds(sid * RPS, RPS)])

    return pl.kernel(
        body,
        out_type=jax.ShapeDtypeStruct((NC, NPAD, 16), f32),
        mesh=_mesh(),
        compiler_params=pltpu.CompilerParams(use_tc_tiling_on_sc=False),
        scratch_types=[
            pltpu.VMEM((K, CH), i32),
            pltpu.VMEM((CH, 16), f32),
            pltpu.VMEM_SHARED((NPAD, 16), f32),
        ],
    )


def _tc_prep(xp, deg2):
    def body(x_ref, dg_ref, d_ref, xs_ref):
        deg = dg_ref[0, :, 0:1] + dg_ref[1, :, 0:1] + 1.0
        d = lax.rsqrt(deg)
        d_ref[...] = d
        xs_ref[...] = x_ref[...] * d

    return pl.pallas_call(
        body,
        out_shape=(jax.ShapeDtypeStruct((NPAD, 1), f32),
                   jax.ShapeDtypeStruct((NPAD, 128), f32)),
    )(xp, deg2)


def _tc_mid1(acc, xs, d, b1, W1, W2):
    def body(a_ref, xs_ref, d_ref, b_ref, w1_ref, w2_ref, out_ref):
        d2 = d_ref[...]
        agg = a_ref[0] + a_ref[1] + xs_ref[...]
        h1 = lax.dot_general(agg, w1_ref[...],
                             (((1,), (0,)), ((), ())), precision=_HIGH)
        x2 = jnp.maximum(h1 * d2 + b_ref[...], 0.0)
        rid = lax.broadcasted_iota(i32, (NPAD, 1), 0)
        x2 = jnp.where(rid < N, x2, 0.0)
        h2 = lax.dot_general(x2, w2_ref[...],
                             (((1,), (0,)), ((), ())), precision=_HIGH)
        out_ref[...] = h2 * d2

    return pl.pallas_call(
        body,
        out_shape=jax.ShapeDtypeStruct((NPAD, 128), f32),
    )(acc, xs, d, b1, W1, W2)


def _tc_mid(acc, hs, d, b, W, Fn):
    def body(a_ref, hs_ref, d_ref, b_ref, w_ref, out_ref):
        d2 = d_ref[...]
        xl = jnp.maximum((a_ref[0] + a_ref[1] + hs_ref[...]) * d2 + b_ref[...], 0.0)
        rid = lax.broadcasted_iota(i32, (NPAD, 1), 0)
        xl = jnp.where(rid < N, xl, 0.0)
        h = lax.dot_general(xl, w_ref[...],
                            (((1,), (0,)), ((), ())), precision=_HIGH)
        out_ref[...] = h * d2

    return pl.pallas_call(
        body,
        out_shape=jax.ShapeDtypeStruct((NPAD, Fn), f32),
    )(acc, hs, d, b, W)


def _tc_final(acc, hs, d, b, batchp, Wc1, bc1, Wc2, bc2):
    def body(a_ref, hs_ref, d_ref, b_ref, bt_ref, w1_ref, b1_ref,
             w2_ref, b2_ref, out_ref):
        d2 = d_ref[...]
        h3 = jnp.maximum((a_ref[0] + a_ref[1] + hs_ref[...]) * d2 + b_ref[...], 0.0)
        rid = lax.broadcasted_iota(i32, (NPAD, 1), 0)
        h3 = jnp.where(rid < N, h3, 0.0)
        gid = lax.broadcasted_iota(i32, (NPAD, 64), 1)
        onehot = (bt_ref[...] == gid).astype(f32)
        cnt = jnp.sum(onehot, axis=0, keepdims=True)
        onehot = onehot / jnp.maximum(cnt, 1.0)
        g = lax.dot_general(onehot, h3,
                            (((0,), (0,)), ((), ())), precision=_HIGH)
        g = jnp.maximum(lax.dot_general(g, w1_ref[...],
                                        (((1,), (0,)), ((), ())),
                                        precision=_HIGH) + b1_ref[...], 0.0)
        logits = lax.dot_general(g, w2_ref[...],
                                 (((1,), (0,)), ((), ())),
                                 precision=_HIGH) + b2_ref[...]
        m = jnp.max(logits, axis=1, keepdims=True)
        lse = jnp.log(jnp.sum(jnp.exp(logits - m), axis=1, keepdims=True)) + m
        out_ref[...] = logits - lse

    return pl.pallas_call(
        body,
        out_shape=jax.ShapeDtypeStruct((64, 10), f32),
    )(acc, hs, d, b, batchp, Wc1, bc1, Wc2, bc2)


def kernel(x, edge_index, batch, W1, b1, W2, b2, W3, b3, Wc1, bc1, Wc2, bc2):
    src = edge_index[0].astype(i32)
    dst = edge_index[1].astype(i32)
    padv = N + (jnp.arange(EPT_PAD - EPT, dtype=i32) % (NPAD - N))
    pad_block = jnp.broadcast_to(padv[None, :], (NW, EPT_PAD - EPT))
    srcp = jnp.concatenate([src.reshape(NW, EPT), pad_block],
                           axis=1).reshape(NW, K, CH)
    dstp = jnp.concatenate([dst.reshape(NW, EPT), pad_block],
                           axis=1).reshape(NW, K, CH)
    xp = jnp.pad(x, ((0, NPAD - N), (0, 0)))
    batchp = jnp.concatenate(
        [batch.astype(i32), jnp.full((NPAD - N,), 64, i32)]).reshape(NPAD, 1)
    z128 = jnp.zeros((NPAD, 128), f32)
    z16 = jnp.zeros((NPAD, 16), f32)
    ones16 = jnp.ones((CH, 16), f32)

    deg2 = _sc_deg()(dstp, z16, ones16)
    d, xs = _tc_prep(xp, deg2)
    acc1 = _sc_scatter(128)(xs, srcp, dstp, z128)
    hs2 = _tc_mid1(acc1, xs, d, b1.reshape(1, -1), W1, W2)
    acc2 = _sc_scatter(128)(hs2, srcp, dstp, z128)
    hs3 = _tc_mid(acc2, hs2, d, b2.reshape(1, -1), W3, 128)
    acc3 = _sc_scatter(128)(hs3, srcp, dstp, z128)
    return _tc_final(acc3, hs3, d, b3.reshape(1, -1), batchp,
                     Wc1, bc1.reshape(1, -1), Wc2, bc2.reshape(1, -1))


# speedup vs baseline: 29.8516x; 1.6052x over previous
import jax
import jax.numpy as jnp
from jax import lax
from jax.experimental import pallas as pl
from jax.experimental.pallas import tpu as pltpu
from jax.experimental.pallas import tpu_sc as plsc

N = 10000
NPAD = 10112
E = 320000
NC, NS = 2, 16
NW = NC * NS
EPT = E // NW
CH = 80
K = 128
EPT_PAD = K * CH
PH = 64
RPS = NPAD // NS

f32 = jnp.float32
i32 = jnp.int32
_HIGH = lax.Precision.HIGHEST


def _mesh():
    return plsc.VectorSubcoreMesh(
        core_axis_name="c", subcore_axis_name="s",
        num_cores=NC, num_subcores=NS)


def _sc_scatter(F, tc_tiling=True):
    def body(table, srcp, dstp, zin, out, idx_s, idx_d, rows, acc,
             gsem, ssem, zsem):
        cid = lax.axis_index("c")
        sid = lax.axis_index("s")
        wid = sid * NC + cid
        zcp = pltpu.async_copy(zin.at[pl.ds(sid * RPS, RPS)],
                               acc.at[pl.ds(sid * RPS, RPS)], zsem)

        def stage(base, cnt):
            pltpu.sync_copy(srcp.at[wid, pl.ds(base, cnt)],
                            idx_s.at[pl.ds(0, cnt)])
            pltpu.sync_copy(dstp.at[wid, pl.ds(base, cnt)],
                            idx_d.at[pl.ds(0, cnt)])
            pltpu.async_copy(table.at[idx_s.at[0]], rows.at[0], gsem.at[0])

        stage(0, PH)
        zcp.wait()
        plsc.subcore_barrier()

        for phase, cnt in ((0, PH), (1, K - PH)):
            if phase:
                stage(PH, cnt)

            def step(j, carry):
                b = lax.rem(j, 3)
                nb = lax.rem(j + 1, 3)

                @pl.when(j + 1 < cnt)
                def _():
                    @pl.when(j >= 2)
                    def _():
                        pltpu.make_async_copy(
                            rows.at[nb], acc.at[idx_d.at[j]],
                            ssem.at[nb]).wait()
                    pltpu.async_copy(table.at[idx_s.at[j + 1]], rows.at[nb],
                                     gsem.at[nb])

                pltpu.make_async_copy(table.at[idx_s.at[j]], rows.at[b],
                                      gsem.at[b]).wait()
                pltpu.async_copy(rows.at[b], acc.at[idx_d.at[j]],
                                 ssem.at[b], add=True)
                return carry

            lax.fori_loop(0, cnt, step, 0)
            for bb in range(3):
                pltpu.make_async_copy(rows.at[bb], acc.at[idx_d.at[0]],
                                      ssem.at[bb]).wait()
        plsc.subcore_barrier()
        pltpu.sync_copy(acc.at[pl.ds(sid * RPS, RPS)],
                        out.at[cid, pl.ds(sid * RPS, RPS)])

    return pl.kernel(
        body,
        out_type=jax.ShapeDtypeStruct((NC, NPAD, F), f32),
        mesh=_mesh(),
        compiler_params=pltpu.CompilerParams(use_tc_tiling_on_sc=tc_tiling),
        scratch_types=[
            pltpu.VMEM((PH, CH), i32),
            pltpu.VMEM((PH, CH), i32),
            pltpu.VMEM((3, CH, F), f32),
            pltpu.VMEM_SHARED((NPAD, F), f32),
            pltpu.SemaphoreType.DMA((3,)),
            pltpu.SemaphoreType.DMA((3,)),
            pltpu.SemaphoreType.DMA,
        ],
    )


def _sc_deg():
    def body(dstp, zin, ones_in, out, idx_d, rows, acc, ssem):
        cid = lax.axis_index("c")
        sid = lax.axis_index("s")
        wid = sid * NC + cid
        pltpu.sync_copy(zin.at[pl.ds(sid * RPS, RPS)],
                        acc.at[pl.ds(sid * RPS, RPS)])
        pltpu.sync_copy(dstp.at[wid], idx_d)
        pltpu.sync_copy(ones_in, rows)
        plsc.subcore_barrier()

        def step(j, carry):
            pltpu.async_copy(rows, acc.at[idx_d.at[j]], ssem, add=True)
            return carry

        lax.fori_loop(0, K, step, 0)

        def drain(j, carry):
            pltpu.make_async_copy(rows, acc.at[idx_d.at[0]], ssem).wait()
            return carry

        lax.fori_loop(0, K, drain, 0)
        plsc.subcore_barrier()
        pltpu.sync_copy(acc.at[pl.ds(sid * RPS, RPS)],
                        out.at[cid, pl.ds(sid * RPS, RPS)])

    return pl.kernel(
        body,
        out_type=jax.ShapeDtypeStruct((NC, NPAD, 16), f32),
        mesh=_mesh(),
        compiler_params=pltpu.CompilerParams(use_tc_tiling_on_sc=False),
        scratch_types=[
            pltpu.VMEM((K, CH), i32),
            pltpu.VMEM((CH, 16), f32),
            pltpu.VMEM_SHARED((NPAD, 16), f32),
            pltpu.SemaphoreType.DMA,
        ],
    )


def _tc_prep(x, W1, deg2):
    def body(x_ref, w_ref, dg_ref, d_ref, hs_ref):
        deg = dg_ref[0, :, 0:1] + dg_ref[1, :, 0:1] + 1.0
        d = lax.rsqrt(deg)
        d_ref[...] = d
        h = lax.dot_general(x_ref[...], w_ref[...],
                            (((1,), (0,)), ((), ())), precision=_HIGH)
        hs_ref[0:N, :] = h * d[0:N]
        hs_ref[N:NPAD, :] = jnp.zeros((NPAD - N, 64), f32)

    return pl.pallas_call(
        body,
        out_shape=(jax.ShapeDtypeStruct((NPAD, 1), f32),
                   jax.ShapeDtypeStruct((NPAD, 64), f32)),
    )(x, W1, deg2)


def _tc_mid1(acc, xs, d, b1, W1, W2):
    def body(a_ref, xs_ref, d_ref, b_ref, w1_ref, w2_ref, out_ref):
        d2 = d_ref[...]
        agg = a_ref[0] + a_ref[1] + xs_ref[...]
        h1 = lax.dot_general(agg, w1_ref[...],
                             (((1,), (0,)), ((), ())), precision=_HIGH)
        x2 = jnp.maximum(h1 * d2 + b_ref[...], 0.0)
        rid = lax.broadcasted_iota(i32, (NPAD, 1), 0)
        x2 = jnp.where(rid < N, x2, 0.0)
        h2 = lax.dot_general(x2, w2_ref[...],
                             (((1,), (0,)), ((), ())), precision=_HIGH)
        out_ref[...] = h2 * d2

    return pl.pallas_call(
        body,
        out_shape=jax.ShapeDtypeStruct((NPAD, 128), f32),
    )(acc, xs, d, b1, W1, W2)


def _tc_mid(acc, hs, d, b, W, Fn):
    def body(a_ref, hs_ref, d_ref, b_ref, w_ref, out_ref):
        d2 = d_ref[...]
        xl = jnp.maximum((a_ref[0] + a_ref[1] + hs_ref[...]) * d2 + b_ref[...], 0.0)
        rid = lax.broadcasted_iota(i32, (NPAD, 1), 0)
        xl = jnp.where(rid < N, xl, 0.0)
        h = lax.dot_general(xl, w_ref[...],
                            (((1,), (0,)), ((), ())), precision=_HIGH)
        out_ref[...] = h * d2

    return pl.pallas_call(
        body,
        out_shape=jax.ShapeDtypeStruct((NPAD, Fn), f32),
    )(acc, hs, d, b, W)


def _tc_final(acc, hs, d, b, batchp, Wc1, bc1, Wc2, bc2):
    def body(a_ref, hs_ref, d_ref, b_ref, bt_ref, w1_ref, b1_ref,
             w2_ref, b2_ref, out_ref):
        d2 = d_ref[0:N]
        h3 = jnp.maximum(
            (a_ref[0, 0:N] + a_ref[1, 0:N] + hs_ref[0:N]) * d2 + b_ref[...],
            0.0)
        gid = lax.broadcasted_iota(i32, (N, 64), 1)
        onehot = (bt_ref[...] == gid).astype(f32)
        cnt = jnp.sum(onehot, axis=0, keepdims=True)
        onehot = onehot / jnp.maximum(cnt, 1.0)
        g = lax.dot_general(onehot, h3,
                            (((0,), (0,)), ((), ())), precision=_HIGH)
        g = jnp.maximum(lax.dot_general(g, w1_ref[...],
                                        (((1,), (0,)), ((), ())),
                                        precision=_HIGH) + b1_ref[...], 0.0)
        logits = lax.dot_general(g, w2_ref[...],
                                 (((1,), (0,)), ((), ())),
                                 precision=_HIGH) + b2_ref[...]
        m = jnp.max(logits, axis=1, keepdims=True)
        lse = jnp.log(jnp.sum(jnp.exp(logits - m), axis=1, keepdims=True)) + m
        out_ref[...] = logits - lse

    return pl.pallas_call(
        body,
        out_shape=jax.ShapeDtypeStruct((64, 10), f32),
    )(acc, hs, d, b, batchp, Wc1, bc1, Wc2, bc2)


def kernel(x, edge_index, batch, W1, b1, W2, b2, W3, b3, Wc1, bc1, Wc2, bc2):
    src = edge_index[0].astype(i32)
    dst = edge_index[1].astype(i32)
    padv = N + (jnp.arange(EPT_PAD - EPT, dtype=i32) % (NPAD - N))
    pad_block = jnp.broadcast_to(padv[None, :], (NW, EPT_PAD - EPT))
    srcp = jnp.concatenate([src.reshape(NW, EPT), pad_block],
                           axis=1).reshape(NW, K, CH)
    dstp = jnp.concatenate([dst.reshape(NW, EPT), pad_block],
                           axis=1).reshape(NW, K, CH)
    batchp = batch.astype(i32).reshape(N, 1)
    z128 = jnp.zeros((NPAD, 128), f32)
    z64 = jnp.zeros((NPAD, 64), f32)
    z16 = jnp.zeros((NPAD, 16), f32)
    ones16 = jnp.ones((CH, 16), f32)

    deg2 = _sc_deg()(dstp, z16, ones16)
    d, hs1 = _tc_prep(x, W1, deg2)
    acc1 = _sc_scatter(64, tc_tiling=False)(hs1, srcp, dstp, z64)
    hs2 = _tc_mid(acc1, hs1, d, b1.reshape(1, -1), W2, 128)
    acc2 = _sc_scatter(128)(hs2, srcp, dstp, z128)
    hs3 = _tc_mid(acc2, hs2, d, b2.reshape(1, -1), W3, 128)
    acc3 = _sc_scatter(128)(hs3, srcp, dstp, z128)
    return _tc_final(acc3, hs3, d, b3.reshape(1, -1), batchp,
                     Wc1, bc1.reshape(1, -1), Wc2, bc2.reshape(1, -1))


# speedup vs baseline: 29.8708x; 1.0006x over previous
import jax
import jax.numpy as jnp
from jax import lax
from jax.experimental import pallas as pl
from jax.experimental.pallas import tpu as pltpu
from jax.experimental.pallas import tpu_sc as plsc

N = 10000
NPAD = 10112
E = 320000
NC, NS = 2, 16
NW = NC * NS
EPT = E // NW
CH = 80
K = 128
EPT_PAD = K * CH
PH = 64
RPS = NPAD // NS

f32 = jnp.float32
i32 = jnp.int32
_HIGH = lax.Precision.HIGHEST


def _mesh():
    return plsc.VectorSubcoreMesh(
        core_axis_name="c", subcore_axis_name="s",
        num_cores=NC, num_subcores=NS)


def _sc_scatter(F, tc_tiling=True):
    def body(table, srcp, dstp, zin, out, idx_s, idx_d, rows, acc,
             gsem, ssem, zsem):
        cid = lax.axis_index("c")
        sid = lax.axis_index("s")
        wid = sid * NC + cid
        zcp = pltpu.async_copy(zin.at[pl.ds(sid * RPS, RPS)],
                               acc.at[pl.ds(sid * RPS, RPS)], zsem)

        def stage(base, cnt):
            pltpu.sync_copy(srcp.at[wid, pl.ds(base, cnt)],
                            idx_s.at[pl.ds(0, cnt)])
            pltpu.sync_copy(dstp.at[wid, pl.ds(base, cnt)],
                            idx_d.at[pl.ds(0, cnt)])
            pltpu.async_copy(table.at[idx_s.at[0]], rows.at[0], gsem.at[0])

        stage(0, PH)
        zcp.wait()
        plsc.subcore_barrier()

        for phase, cnt in ((0, PH), (1, K - PH)):
            if phase:
                stage(PH, cnt)

            def step(j, carry):
                b = lax.rem(j, 3)
                nb = lax.rem(j + 1, 3)

                @pl.when(j + 1 < cnt)
                def _():
                    @pl.when(j >= 2)
                    def _():
                        pltpu.make_async_copy(
                            rows.at[nb], acc.at[idx_d.at[j]],
                            ssem.at[nb]).wait()
                    pltpu.async_copy(table.at[idx_s.at[j + 1]], rows.at[nb],
                                     gsem.at[nb])

                pltpu.make_async_copy(table.at[idx_s.at[j]], rows.at[b],
                                      gsem.at[b]).wait()
                pltpu.async_copy(rows.at[b], acc.at[idx_d.at[j]],
                                 ssem.at[b], add=True)
                return carry

            lax.fori_loop(0, cnt, step, 0)
            for bb in range(3):
                pltpu.make_async_copy(rows.at[bb], acc.at[idx_d.at[0]],
                                      ssem.at[bb]).wait()
        plsc.subcore_barrier()
        pltpu.sync_copy(acc.at[pl.ds(sid * RPS, RPS)],
                        out.at[cid, pl.ds(sid * RPS, RPS)])

    return pl.kernel(
        body,
        out_type=jax.ShapeDtypeStruct((NC, NPAD, F), f32),
        mesh=_mesh(),
        compiler_params=pltpu.CompilerParams(use_tc_tiling_on_sc=tc_tiling),
        scratch_types=[
            pltpu.VMEM((PH, CH), i32),
            pltpu.VMEM((PH, CH), i32),
            pltpu.VMEM((3, CH, F), f32),
            pltpu.VMEM_SHARED((NPAD, F), f32),
            pltpu.SemaphoreType.DMA((3,)),
            pltpu.SemaphoreType.DMA((3,)),
            pltpu.SemaphoreType.DMA,
        ],
    )


def _sc_deg():
    def body(dstp, zin, ones_in, out, idx_d, rows, acc, ssem):
        cid = lax.axis_index("c")
        sid = lax.axis_index("s")
        wid = sid * NC + cid
        pltpu.sync_copy(zin.at[pl.ds(sid * RPS, RPS)],
                        acc.at[pl.ds(sid * RPS, RPS)])
        pltpu.sync_copy(dstp.at[wid], idx_d)
        pltpu.sync_copy(ones_in, rows)
        plsc.subcore_barrier()

        def step(j, carry):
            pltpu.async_copy(rows, acc.at[idx_d.at[j]], ssem, add=True)
            return carry

        lax.fori_loop(0, K, step, 0)

        def drain(j, carry):
            pltpu.make_async_copy(rows, acc.at[idx_d.at[0]], ssem).wait()
            return carry

        lax.fori_loop(0, K, drain, 0)
        plsc.subcore_barrier()
        pltpu.sync_copy(acc.at[pl.ds(sid * RPS, RPS)],
                        out.at[cid, pl.ds(sid * RPS, RPS)])

    return pl.kernel(
        body,
        out_type=jax.ShapeDtypeStruct((NC, NPAD, 16), f32),
        mesh=_mesh(),
        compiler_params=pltpu.CompilerParams(use_tc_tiling_on_sc=False),
        scratch_types=[
            pltpu.VMEM((K, CH), i32),
            pltpu.VMEM((CH, 16), f32),
            pltpu.VMEM_SHARED((NPAD, 16), f32),
            pltpu.SemaphoreType.DMA,
        ],
    )


def _tc_prep(x, W1, deg2):
    def body(x_ref, w_ref, dg_ref, d_ref, hs_ref):
        deg = dg_ref[0, :, 0:1] + dg_ref[1, :, 0:1] + 1.0
        d = lax.rsqrt(deg)
        d_ref[...] = d
        h = lax.dot_general(x_ref[...], w_ref[...],
                            (((1,), (0,)), ((), ())), precision=_HIGH)
        hs_ref[0:N, :] = h * d[0:N]
        hs_ref[N:NPAD, :] = jnp.zeros((NPAD - N, 64), f32)

    return pl.pallas_call(
        body,
        out_shape=(jax.ShapeDtypeStruct((NPAD, 1), f32),
                   jax.ShapeDtypeStruct((NPAD, 64), f32)),
    )(x, W1, deg2)


def _tc_mid(acc, hs, d, b, W, Fn):
    def body(a_ref, hs_ref, d_ref, b_ref, w_ref, out_ref):
        d2 = d_ref[...]
        xl = jnp.maximum((a_ref[0] + a_ref[1] + hs_ref[...]) * d2 + b_ref[...], 0.0)
        rid = lax.broadcasted_iota(i32, (NPAD, 1), 0)
        xl = jnp.where(rid < N, xl, 0.0)
        h = lax.dot_general(xl, w_ref[...],
                            (((1,), (0,)), ((), ())), precision=_HIGH)
        out_ref[...] = h * d2

    return pl.pallas_call(
        body,
        out_shape=jax.ShapeDtypeStruct((NPAD, Fn), f32),
    )(acc, hs, d, b, W)


def _tc_final(acc, hs, d, b, batchp, Wc1, bc1, Wc2, bc2):
    def body(a_ref, hs_ref, d_ref, b_ref, bt_ref, w1_ref, b1_ref,
             w2_ref, b2_ref, out_ref):
        d2 = d_ref[0:N]
        h3 = jnp.maximum(
            (a_ref[0, 0:N] + a_ref[1, 0:N] + hs_ref[0:N]) * d2 + b_ref[...],
            0.0)
        gid = lax.broadcasted_iota(i32, (N, 64), 1)
        onehot = (bt_ref[...] == gid).astype(f32)
        cnt = jnp.sum(onehot, axis=0, keepdims=True)
        onehot = onehot / jnp.maximum(cnt, 1.0)
        g = lax.dot_general(onehot, h3,
                            (((0,), (0,)), ((), ())), precision=_HIGH)
        g = jnp.maximum(lax.dot_general(g, w1_ref[...],
                                        (((1,), (0,)), ((), ())),
                                        precision=_HIGH) + b1_ref[...], 0.0)
        logits = lax.dot_general(g, w2_ref[...],
                                 (((1,), (0,)), ((), ())),
                                 precision=_HIGH) + b2_ref[...]
        m = jnp.max(logits, axis=1, keepdims=True)
        lse = jnp.log(jnp.sum(jnp.exp(logits - m), axis=1, keepdims=True)) + m
        out_ref[...] = logits - lse

    return pl.pallas_call(
        body,
        out_shape=jax.ShapeDtypeStruct((64, 10), f32),
    )(acc, hs, d, b, batchp, Wc1, bc1, Wc2, bc2)


def kernel(x, edge_index, batch, W1, b1, W2, b2, W3, b3, Wc1, bc1, Wc2, bc2):
    src = edge_index[0].astype(i32)
    dst = edge_index[1].astype(i32)
    padv = N + (jnp.arange(EPT_PAD - EPT, dtype=i32) % (NPAD - N))
    pad_block = jnp.broadcast_to(padv[None, :], (NW, EPT_PAD - EPT))
    srcp = jnp.concatenate([src.reshape(NW, EPT), pad_block],
                           axis=1).reshape(NW, K, CH)
    dstp = jnp.concatenate([dst.reshape(NW, EPT), pad_block],
                           axis=1).reshape(NW, K, CH)
    batchp = batch.astype(i32).reshape(N, 1)
    z128 = jnp.zeros((NPAD, 128), f32)
    z64 = jnp.zeros((NPAD, 64), f32)
    z16 = jnp.zeros((NPAD, 16), f32)
    ones16 = jnp.ones((CH, 16), f32)

    deg2 = _sc_deg()(dstp, z16, ones16)
    d, hs1 = _tc_prep(x, W1, deg2)
    acc1 = _sc_scatter(64, tc_tiling=False)(hs1, srcp, dstp, z64)
    hs2 = _tc_mid(acc1, hs1, d, b1.reshape(1, -1), W2, 128)
    acc2 = _sc_scatter(128)(hs2, srcp, dstp, z128)
    hs3 = _tc_mid(acc2, hs2, d, b2.reshape(1, -1), W3, 128)
    acc3 = _sc_scatter(128)(hs3, srcp, dstp, z128)
    return _tc_final(acc3, hs3, d, b3.reshape(1, -1), batchp,
                     Wc1, bc1.reshape(1, -1), Wc2, bc2.reshape(1, -1))
